# baseline (speedup 1.0000x reference)
import functools

import jax
import jax.numpy as jnp
from jax import lax
from jax.experimental import pallas as pl
from jax.experimental.pallas import tpu as pltpu
from jax.experimental.pallas import tpu_sc as plsc

_NW = 32
_C = 512
_IB = 128
_NJ = _C // _IB

_FACE_BYTES = 3 * _NJ * _IB * 4
_VERT_BYTES = 3 * _NJ * _IB * 16 * 4
_OUT_BYTES = _C * 16 * 4


def _sc_body(fim_hbm, w_hbm, fv0_hbm, fv1_hbm, fv2_hbm, va_hbm, out_hbm,
             fim_a, fim_b, fvd_a, fvd_b, at_a, at_b, wv_a, wv_b, ov_a, ov_b,
             dumm, sem_fa, sem_fb, sem_va, sem_vb, sem_oa, sem_ob,
             sem_la, sem_lb):
    n_chunks = fim_hbm.shape[0] * _IB // _C // _NW
    n2 = n_chunks // 2
    pix_per_w = n_chunks * _C
    wid = lax.axis_index("s") * 2 + lax.axis_index("c")
    batch_pix = 512 * 512
    off = (wid * pix_per_w // batch_pix) * jnp.int32(200000)

    def prefetch_in(t, fim_v, wv, sem_l):
        base_pix = wid * pix_per_w + t * _C
        base_row = base_pix // _IB
        pltpu.async_copy(fim_hbm.at[pl.ds(base_row, _NJ)], fim_v, sem_l)
        pltpu.async_copy(w_hbm.at[:, pl.ds(base_pix, _C)], wv, sem_l)

    def prepare(fim_v, wv, sem_l, sem_f, fvd):
        pltpu.make_async_copy(fim_hbm.at[pl.ds(0, _NJ)], fim_v, sem_l).wait()
        pltpu.make_async_copy(w_hbm.at[:, pl.ds(0, _C)], wv, sem_l).wait()

        def _offs(i):
            r = i // (_IB // 16)
            c = (i % (_IB // 16)) * 16
            fim_v[r, pl.ds(c, 16)] = fim_v[r, pl.ds(c, 16)] + off

        plsc.parallel_loop(0, _C // 16, 1, unroll=4)(_offs)
        for j in range(_NJ):
            pltpu.async_copy(fv0_hbm.at[fim_v.at[j]], fvd[0].at[j], sem_f)
            pltpu.async_copy(fv1_hbm.at[fim_v.at[j]], fvd[1].at[j], sem_f)
            pltpu.async_copy(fv2_hbm.at[fim_v.at[j]], fvd[2].at[j], sem_f)

    def fire_vertex(fvd, at, sem_v):
        for j in range(_NJ):
            for k in range(3):
                pltpu.async_copy(va_hbm.at[fvd[k].at[j]],
                                 at[k].at[pl.ds(j * _IB, _IB)], sem_v)

    def drain_face(fim_v, fvd, sem_f):
        for j in range(_NJ):
            pltpu.make_async_copy(fv0_hbm.at[fim_v.at[j]], fvd[0].at[j], sem_f).wait()
            pltpu.make_async_copy(fv1_hbm.at[fim_v.at[j]], fvd[1].at[j], sem_f).wait()
            pltpu.make_async_copy(fv2_hbm.at[fim_v.at[j]], fvd[2].at[j], sem_f).wait()

    def drain_vertex(fvd, at, sem_v):
        for j in range(_NJ):
            for k in range(3):
                pltpu.make_async_copy(va_hbm.at[fvd[k].at[j]],
                                      at[k].at[pl.ds(j * _IB, _IB)], sem_v).wait()

    def drain_out(ov, sem_o):
        for a in range(16):
            pltpu.make_async_copy(ov.at[a, pl.ds(0, _C)], out_hbm.at[0, 0, a], sem_o).wait()

    iota16 = lax.iota(jnp.int32, 16)

    def compute_out(t, tt, at, wv, ov, sem_o):
        a0, a1, a2 = at

        def grp(g):
            p0 = g * 16
            w0v = wv[0, pl.ds(p0, 16)]
            w1v = wv[1, pl.ds(p0, 16)]
            w2v = wv[2, pl.ds(p0, 16)]
            for q in range(16):
                p = p0 + q
                acc = (a0[p, :] * w0v[q] + a1[p, :] * w1v[q]
                       + a2[p, :] * w2v[q])
                plsc.store_scatter(ov, [iota16, jnp.full((16,), p, jnp.int32)], acc)

        @pl.when(tt > 0)
        def _():
            drain_out(ov, sem_o)

        plsc.parallel_loop(0, _C // 16, 1, unroll=8)(grp)
        base_pix = wid * pix_per_w + t * _C
        n_img = base_pix // batch_pix
        h_img = (base_pix - n_img * batch_pix) // 512
        for a in range(16):
            pltpu.async_copy(ov.at[a, pl.ds(0, _C)], out_hbm.at[n_img, h_img, a], sem_o)

    prefetch_in(0, fim_a, wv_a, sem_la)
    prefetch_in(1, fim_b, wv_b, sem_lb)
    prepare(fim_a, wv_a, sem_la, sem_fa, fvd_a)

    def body2(tt, carry):
        ta = 2 * tt
        drain_face(fim_a, fvd_a, sem_fa)
        fire_vertex(fvd_a, at_a, sem_va)
        prepare(fim_b, wv_b, sem_lb, sem_fb, fvd_b)
        drain_vertex(fvd_a, at_a, sem_va)
        compute_out(ta, tt, at_a, wv_a, ov_a, sem_oa)

        @pl.when(tt + 1 < n2)
        def _():
            prefetch_in(ta + 2, fim_a, wv_a, sem_la)

        drain_face(fim_b, fvd_b, sem_fb)
        fire_vertex(fvd_b, at_b, sem_vb)
        drain_vertex(fvd_b, at_b, sem_vb)
        compute_out(ta + 1, tt, at_b, wv_b, ov_b, sem_ob)

        @pl.when(tt + 1 < n2)
        def _():
            prefetch_in(ta + 3, fim_b, wv_b, sem_lb)
            prepare(fim_a, wv_a, sem_la, sem_fa, fvd_a)

        return carry

    lax.fori_loop(0, n2, body2, 0)
    drain_out(ov_a, sem_oa)
    drain_out(ov_b, sem_ob)


def kernel(v_attr, faces_v_idx, face_index_map, weight_map):
    N, H, W = face_index_map.shape
    V, A = v_attr.shape[1], v_attr.shape[2]
    F = faces_v_idx.shape[1]
    P = N * H * W

    fim = face_index_map.astype(jnp.int32).reshape(P // _IB, _IB)
    fv = faces_v_idx.astype(jnp.int32).reshape(N * F, 3)
    fv0, fv1, fv2 = fv[:, 0], fv[:, 1], fv[:, 2]
    wm = weight_map.transpose(3, 4, 0, 1, 2).reshape(3, P)
    va = v_attr.reshape(V, A)

    idx_t = pltpu.VMEM((_NJ, _IB), jnp.int32)
    att_t = pltpu.VMEM((_C, 16), jnp.float32)

    mesh = plsc.VectorSubcoreMesh(core_axis_name="c", subcore_axis_name="s")
    run = pl.kernel(
        _sc_body,
        mesh=mesh,
        compiler_params=pltpu.CompilerParams(use_tc_tiling_on_sc=False,
                                             needs_layout_passes=False),
        out_type=jax.ShapeDtypeStruct((N, H, A, W), jnp.float32),
        scratch_types=[
            idx_t, idx_t,
            [idx_t, idx_t, idx_t],
            [idx_t, idx_t, idx_t],
            [att_t, att_t, att_t],
            [att_t, att_t, att_t],
            pltpu.VMEM((3, _C), jnp.float32),
            pltpu.VMEM((3, _C), jnp.float32),
            pltpu.VMEM((16, _C + 1), jnp.float32),
            pltpu.VMEM((16, _C + 1), jnp.float32),
            pltpu.VMEM((_C,), jnp.float32),
            pltpu.SemaphoreType.DMA,
            pltpu.SemaphoreType.DMA,
            pltpu.SemaphoreType.DMA,
            pltpu.SemaphoreType.DMA,
            pltpu.SemaphoreType.DMA,
            pltpu.SemaphoreType.DMA,
            pltpu.SemaphoreType.DMA,
            pltpu.SemaphoreType.DMA,
        ],
    )
    return run(fim, wm, fv0, fv1, fv2, va).transpose(0, 1, 3, 2)


# speedup vs baseline: 1.1746x; 1.1746x over previous
import functools

import jax
import jax.numpy as jnp
from jax import lax
from jax.experimental import pallas as pl
from jax.experimental.pallas import tpu as pltpu
from jax.experimental.pallas import tpu_sc as plsc

_NW = 32
_C = 512
_IB = 128
_NJ = _C // _IB

_FACE_BYTES = 3 * _NJ * _IB * 4
_VERT_BYTES = 3 * _NJ * _IB * 16 * 4
_OUT_BYTES = _C * 16 * 4


def _sc_body(fim_hbm, w_hbm, fv0_hbm, fv1_hbm, fv2_hbm, va_hbm, out_hbm,
             fim_a, fim_b, fvd_a, fvd_b, at_a, at_b, wv_a, wv_b, ov_a, ov_b,
             dumm, sem_fa, sem_fb, sem_va, sem_vb, sem_oa, sem_ob):
    n_chunks = fim_hbm.shape[0] * _IB // _C // _NW
    n2 = n_chunks // 2
    pix_per_w = n_chunks * _C
    wid = lax.axis_index("s") * 2 + lax.axis_index("c")
    batch_pix = 512 * 512
    off = (wid * pix_per_w // batch_pix) * jnp.int32(200000)

    def stage_in(t, fim_v, wv, sem_f, fvd):
        base_pix = wid * pix_per_w + t * _C
        base_row = base_pix // _IB
        pltpu.sync_copy(fim_hbm.at[pl.ds(base_row, _NJ)], fim_v)
        pltpu.sync_copy(w_hbm.at[:, pl.ds(base_pix, _C)], wv)
        def _offs(i):
            r = i // (_IB // 16)
            c = (i % (_IB // 16)) * 16
            fim_v[r, pl.ds(c, 16)] = fim_v[r, pl.ds(c, 16)] + off

        plsc.parallel_loop(0, _C // 16, 1, unroll=4)(_offs)
        for j in range(_NJ):
            pltpu.async_copy(fv0_hbm.at[fim_v.at[j]], fvd[0].at[j], sem_f)
            pltpu.async_copy(fv1_hbm.at[fim_v.at[j]], fvd[1].at[j], sem_f)
            pltpu.async_copy(fv2_hbm.at[fim_v.at[j]], fvd[2].at[j], sem_f)

    def fire_vertex(fvd, at, sem_v):
        for j in range(_NJ):
            for k in range(3):
                pltpu.async_copy(va_hbm.at[fvd[k].at[j]],
                                 at[k].at[pl.ds(j * _IB, _IB)], sem_v)

    def drain_face(fim_v, fvd, sem_f):
        for j in range(_NJ):
            pltpu.make_async_copy(fv0_hbm.at[fim_v.at[j]], fvd[0].at[j], sem_f).wait()
            pltpu.make_async_copy(fv1_hbm.at[fim_v.at[j]], fvd[1].at[j], sem_f).wait()
            pltpu.make_async_copy(fv2_hbm.at[fim_v.at[j]], fvd[2].at[j], sem_f).wait()

    def drain_vertex(fvd, at, sem_v):
        for j in range(_NJ):
            for k in range(3):
                pltpu.make_async_copy(va_hbm.at[fvd[k].at[j]],
                                      at[k].at[pl.ds(j * _IB, _IB)], sem_v).wait()

    def drain_out(ov, sem_o):
        for a in range(16):
            pltpu.make_async_copy(ov.at[a, pl.ds(0, _C)], out_hbm.at[0, 0, a], sem_o).wait()

    iota16 = lax.iota(jnp.int32, 16)

    def compute_out(t, tt, at, wv, ov, sem_o):
        a0, a1, a2 = at

        def grp(g):
            p0 = g * 16
            w0v = wv[0, pl.ds(p0, 16)]
            w1v = wv[1, pl.ds(p0, 16)]
            w2v = wv[2, pl.ds(p0, 16)]
            for q in range(16):
                p = p0 + q
                acc = (a0[p, :] * w0v[q] + a1[p, :] * w1v[q]
                       + a2[p, :] * w2v[q])
                plsc.store_scatter(ov, [iota16, jnp.full((16,), p, jnp.int32)], acc)

        @pl.when(tt > 0)
        def _():
            drain_out(ov, sem_o)

        plsc.parallel_loop(0, _C // 16, 1, unroll=8)(grp)
        base_pix = wid * pix_per_w + t * _C
        n_img = base_pix // batch_pix
        h_img = (base_pix - n_img * batch_pix) // 512
        for a in range(16):
            pltpu.async_copy(ov.at[a, pl.ds(0, _C)], out_hbm.at[n_img, h_img, a], sem_o)

    stage_in(0, fim_a, wv_a, sem_fa, fvd_a)

    def body2(tt, carry):
        ta = 2 * tt
        drain_face(fim_a, fvd_a, sem_fa)
        fire_vertex(fvd_a, at_a, sem_va)
        stage_in(ta + 1, fim_b, wv_b, sem_fb, fvd_b)
        drain_vertex(fvd_a, at_a, sem_va)
        compute_out(ta, tt, at_a, wv_a, ov_a, sem_oa)
        drain_face(fim_b, fvd_b, sem_fb)
        fire_vertex(fvd_b, at_b, sem_vb)

        @pl.when(tt + 1 < n2)
        def _():
            stage_in(ta + 2, fim_a, wv_a, sem_fa, fvd_a)

        drain_vertex(fvd_b, at_b, sem_vb)
        compute_out(ta + 1, tt, at_b, wv_b, ov_b, sem_ob)
        return carry

    lax.fori_loop(0, n2, body2, 0)
    drain_out(ov_a, sem_oa)
    drain_out(ov_b, sem_ob)


def kernel(v_attr, faces_v_idx, face_index_map, weight_map):
    N, H, W = face_index_map.shape
    V, A = v_attr.shape[1], v_attr.shape[2]
    F = faces_v_idx.shape[1]
    P = N * H * W

    fim = face_index_map.astype(jnp.int32).reshape(P // _IB, _IB)
    fv = faces_v_idx.astype(jnp.int32).reshape(N * F, 3)
    fv0, fv1, fv2 = fv[:, 0], fv[:, 1], fv[:, 2]
    wm = weight_map.transpose(3, 4, 0, 1, 2).reshape(3, P)
    va = v_attr.reshape(V, A)

    idx_t = pltpu.VMEM((_NJ, _IB), jnp.int32)
    att_t = pltpu.VMEM((_C, 16), jnp.float32)

    mesh = plsc.VectorSubcoreMesh(core_axis_name="c", subcore_axis_name="s")
    run = pl.kernel(
        _sc_body,
        mesh=mesh,
        compiler_params=pltpu.CompilerParams(use_tc_tiling_on_sc=False,
                                             needs_layout_passes=False),
        out_type=jax.ShapeDtypeStruct((N, H, A, W), jnp.float32),
        scratch_types=[
            idx_t, idx_t,
            [idx_t, idx_t, idx_t],
            [idx_t, idx_t, idx_t],
            [att_t, att_t, att_t],
            [att_t, att_t, att_t],
            pltpu.VMEM((3, _C), jnp.float32),
            pltpu.VMEM((3, _C), jnp.float32),
            pltpu.VMEM((16, _C + 1), jnp.float32),
            pltpu.VMEM((16, _C + 1), jnp.float32),
            pltpu.VMEM((_C,), jnp.float32),
            pltpu.SemaphoreType.DMA,
            pltpu.SemaphoreType.DMA,
            pltpu.SemaphoreType.DMA,
            pltpu.SemaphoreType.DMA,
            pltpu.SemaphoreType.DMA,
            pltpu.SemaphoreType.DMA,
        ],
    )
    return run(fim, wm, fv0, fv1, fv2, va).transpose(0, 1, 3, 2)


# speedup vs baseline: 1.1955x; 1.0178x over previous
import functools

import jax
import jax.numpy as jnp
from jax import lax
from jax.experimental import pallas as pl
from jax.experimental.pallas import tpu as pltpu
from jax.experimental.pallas import tpu_sc as plsc

_NW = 32
_C = 512
_IB = 128
_NJ = _C // _IB

_FACE_BYTES = 3 * _NJ * _IB * 4
_VERT_BYTES = 3 * _NJ * _IB * 16 * 4
_OUT_BYTES = _C * 16 * 4


def _sc_body(fim_hbm, w_hbm, fv0_hbm, fv1_hbm, fv2_hbm, va_hbm, out_hbm,
             fim_a, fim_b, fvd_a, fvd_b, at_a, at_b, wv_a, wv_b, ov_a, ov_b,
             dumm, sem_fa, sem_fb, sem_va, sem_vb, sem_oa, sem_ob):
    n_chunks = fim_hbm.shape[0] * _IB // _C // _NW
    n2 = n_chunks // 2
    pix_per_w = n_chunks * _C
    wid = lax.axis_index("s") * 2 + lax.axis_index("c")
    batch_pix = 512 * 512
    off = (wid * pix_per_w // batch_pix) * jnp.int32(200000)

    def stage_in(t, fim_v, wv, sem_f, fvd):
        base_pix = wid * pix_per_w + t * _C
        base_row = base_pix // _IB
        pltpu.sync_copy(fim_hbm.at[pl.ds(base_row, _NJ)], fim_v)
        pltpu.sync_copy(w_hbm.at[:, pl.ds(base_pix, _C)], wv)
        def _offs(i):
            r = i // (_IB // 16)
            c = (i % (_IB // 16)) * 16
            fim_v[r, pl.ds(c, 16)] = fim_v[r, pl.ds(c, 16)] + off

        plsc.parallel_loop(0, _C // 16, 1, unroll=4)(_offs)
        for j in range(_NJ):
            pltpu.async_copy(fv0_hbm.at[fim_v.at[j]], fvd[0].at[j], sem_f)
            pltpu.async_copy(fv1_hbm.at[fim_v.at[j]], fvd[1].at[j], sem_f)
            pltpu.async_copy(fv2_hbm.at[fim_v.at[j]], fvd[2].at[j], sem_f)

    def fire_vertex(fvd, at, sem_v):
        for j in range(_NJ):
            for k in range(3):
                pltpu.async_copy(va_hbm.at[fvd[k].at[j]],
                                 at[k].at[pl.ds(j * _IB, _IB)], sem_v)

    def drain_face(fim_v, fvd, sem_f):
        for j in range(_NJ):
            pltpu.make_async_copy(fv0_hbm.at[fim_v.at[j]], fvd[0].at[j], sem_f).wait()
            pltpu.make_async_copy(fv1_hbm.at[fim_v.at[j]], fvd[1].at[j], sem_f).wait()
            pltpu.make_async_copy(fv2_hbm.at[fim_v.at[j]], fvd[2].at[j], sem_f).wait()

    def drain_vertex(fvd, at, sem_v):
        for j in range(_NJ):
            for k in range(3):
                pltpu.make_async_copy(va_hbm.at[fvd[k].at[j]],
                                      at[k].at[pl.ds(j * _IB, _IB)], sem_v).wait()

    def drain_out(ov, sem_o):
        for a in range(16):
            pltpu.make_async_copy(ov.at[a, pl.ds(0, _C)], out_hbm.at[0, 0, a], sem_o).wait()

    iota16 = lax.iota(jnp.int32, 16)

    def compute_out(t, tt, at, wv, ov, sem_o):
        a0, a1, a2 = at

        def grp(g):
            p0 = g * 16
            w0v = wv[0, pl.ds(p0, 16)]
            w1v = wv[1, pl.ds(p0, 16)]
            w2v = wv[2, pl.ds(p0, 16)]
            for q in range(16):
                p = p0 + q
                acc = (a0[p, :] * w0v[q] + a1[p, :] * w1v[q]
                       + a2[p, :] * w2v[q])
                plsc.store_scatter(ov, [iota16, jnp.full((16,), p, jnp.int32)], acc)

        @pl.when(tt > 0)
        def _():
            drain_out(ov, sem_o)

        plsc.parallel_loop(0, _C // 16, 1, unroll=4)(grp)
        base_pix = wid * pix_per_w + t * _C
        n_img = base_pix // batch_pix
        h_img = (base_pix - n_img * batch_pix) // 512
        for a in range(16):
            pltpu.async_copy(ov.at[a, pl.ds(0, _C)], out_hbm.at[n_img, h_img, a], sem_o)

    stage_in(0, fim_a, wv_a, sem_fa, fvd_a)

    def body2(tt, carry):
        ta = 2 * tt
        drain_face(fim_a, fvd_a, sem_fa)
        fire_vertex(fvd_a, at_a, sem_va)
        stage_in(ta + 1, fim_b, wv_b, sem_fb, fvd_b)
        drain_vertex(fvd_a, at_a, sem_va)
        compute_out(ta, tt, at_a, wv_a, ov_a, sem_oa)
        drain_face(fim_b, fvd_b, sem_fb)
        fire_vertex(fvd_b, at_b, sem_vb)

        @pl.when(tt + 1 < n2)
        def _():
            stage_in(ta + 2, fim_a, wv_a, sem_fa, fvd_a)

        drain_vertex(fvd_b, at_b, sem_vb)
        compute_out(ta + 1, tt, at_b, wv_b, ov_b, sem_ob)
        return carry

    lax.fori_loop(0, n2, body2, 0)
    drain_out(ov_a, sem_oa)
    drain_out(ov_b, sem_ob)


def kernel(v_attr, faces_v_idx, face_index_map, weight_map):
    N, H, W = face_index_map.shape
    V, A = v_attr.shape[1], v_attr.shape[2]
    F = faces_v_idx.shape[1]
    P = N * H * W

    fim = face_index_map.astype(jnp.int32).reshape(P // _IB, _IB)
    fv = faces_v_idx.astype(jnp.int32).reshape(N * F, 3)
    fv0, fv1, fv2 = fv[:, 0], fv[:, 1], fv[:, 2]
    wm = weight_map.transpose(3, 4, 0, 1, 2).reshape(3, P)
    va = v_attr.reshape(V, A)

    idx_t = pltpu.VMEM((_NJ, _IB), jnp.int32)
    att_t = pltpu.VMEM((_C, 16), jnp.float32)

    mesh = plsc.VectorSubcoreMesh(core_axis_name="c", subcore_axis_name="s")
    run = pl.kernel(
        _sc_body,
        mesh=mesh,
        compiler_params=pltpu.CompilerParams(use_tc_tiling_on_sc=False,
                                             needs_layout_passes=False),
        out_type=jax.ShapeDtypeStruct((N, H, A, W), jnp.float32),
        scratch_types=[
            idx_t, idx_t,
            [idx_t, idx_t, idx_t],
            [idx_t, idx_t, idx_t],
            [att_t, att_t, att_t],
            [att_t, att_t, att_t],
            pltpu.VMEM((3, _C), jnp.float32),
            pltpu.VMEM((3, _C), jnp.float32),
            pltpu.VMEM((16, _C + 1), jnp.float32),
            pltpu.VMEM((16, _C + 1), jnp.float32),
            pltpu.VMEM((_C,), jnp.float32),
            pltpu.SemaphoreType.DMA,
            pltpu.SemaphoreType.DMA,
            pltpu.SemaphoreType.DMA,
            pltpu.SemaphoreType.DMA,
            pltpu.SemaphoreType.DMA,
            pltpu.SemaphoreType.DMA,
        ],
    )
    return run(fim, wm, fv0, fv1, fv2, va).transpose(0, 1, 3, 2)
